# baseline (speedup 1.0000x reference)
import functools

import jax
import jax.numpy as jnp
from jax import lax
from jax.experimental import pallas as pl
from jax.experimental.pallas import tpu as pltpu
from jax.experimental.pallas import tpu_sc as plsc

_BLK = 512


def _sc_gather_row(idx_arr, table):
    D = table.shape[1]
    mesh = plsc.VectorSubcoreMesh(core_axis_name="c", subcore_axis_name="s")

    @functools.partial(
        pl.kernel,
        mesh=mesh,
        out_type=jax.ShapeDtypeStruct((1, D), jnp.float32),
        scratch_types=[
            pltpu.VMEM((1,), jnp.int32),
            pltpu.VMEM((1, D), jnp.float32),
            pltpu.SemaphoreType.DMA,
        ],
    )
    def gather(idx_hbm, table_hbm, row_hbm, idx_v, row_v, sem):
        first = (lax.axis_index("c") == 0) & (lax.axis_index("s") == 0)

        @pl.when(first)
        def _():
            pltpu.sync_copy(idx_hbm, idx_v)
            pltpu.async_copy(table_hbm.at[idx_v], row_v, sem).wait()
            pltpu.sync_copy(row_v, row_hbm)

    return gather(idx_arr, table)


def _add_body(row_ref, x_ref, o_ref):
    o_ref[...] = x_ref[...] + row_ref[...]


def kernel(features, layer_idx, layer_embeddings):
    B, S, D = features.shape
    M = B * S
    x2d = features.reshape(M, D)
    idx_arr = jnp.asarray(layer_idx, dtype=jnp.int32).reshape(1)
    row = _sc_gather_row(idx_arr, layer_embeddings)
    out = pl.pallas_call(
        _add_body,
        grid=(M // _BLK,),
        in_specs=[
            pl.BlockSpec((1, D), lambda i: (0, 0)),
            pl.BlockSpec((_BLK, D), lambda i: (i, 0)),
        ],
        out_specs=pl.BlockSpec((_BLK, D), lambda i: (i, 0)),
        out_shape=jax.ShapeDtypeStruct((M, D), jnp.float32),
        compiler_params=pltpu.CompilerParams(
            dimension_semantics=("parallel",),
        ),
    )(row, x2d)
    return out.reshape(B, S, D)
